# baseline (speedup 1.0000x reference)
import functools

import jax
import jax.numpy as jnp
from jax import lax
from jax.experimental import pallas as pl
from jax.experimental.pallas import tpu as pltpu
from jax.experimental.pallas import tpu_sc as plsc

N = 10000
E = 320000
D = 128
NPAD = 10240
K = 128
NC, NS = 2, 16
NW = NC * NS
CPT = 80
EPAD = NW * CPT * K
ROWS_PER_TILE = NPAD // NS
DW = 128


NB = 2
WSZ = 16
CPT0 = 32
CPT1 = 2 * CPT - CPT0


def _segsum_body(g_hbm, srcs_hbm, dsts_hbm, zer_hbm,
                 part_hbm,
                 srcw, dstw, rows0, rows1, acc_sh,
                 wsem, sem0, sem1):
    rows = (rows0, rows1)
    sems = (sem0, sem1)
    c = lax.axis_index("c")
    s = lax.axis_index("s")
    wid = s * NC + c

    r0 = s * ROWS_PER_TILE
    pltpu.sync_copy(zer_hbm.at[pl.ds(r0, ROWS_PER_TILE)],
                    acc_sh.at[pl.ds(r0, ROWS_PER_TILE)])

    plsc.subcore_barrier()

    def _start(t, b):
        pltpu.async_copy(g_hbm.at[srcw.at[t]], rows[b], sems[b])

    def _wait(b):
        pltpu.make_async_copy(g_hbm.at[pl.ds(0, K)], rows[b], sems[b]).wait()

    base = jnp.where(c == 0, s * CPT0, NS * CPT0 + s * CPT1)
    nwin = jnp.where(c == 0, CPT0 // WSZ, CPT1 // WSZ)

    @pl.loop(0, nwin)
    def _win(w):
        vec = base + w * WSZ + lax.iota(jnp.int32, WSZ)
        pltpu.async_copy(srcs_hbm.at[vec], srcw, wsem).wait()
        pltpu.async_copy(dsts_hbm.at[vec], dstw, wsem).wait()

        for b in range(NB):
            _start(b, b)

        @pl.loop(0, WSZ // NB - 1)
        def _group(g):
            for b in range(NB):
                t = g * NB + b
                _wait(b)
                pltpu.sync_copy(rows[b], acc_sh.at[dstw.at[t]], add=True)
                _start(t + NB, b)

        for b in range(NB):
            t = WSZ - NB + b
            _wait(b)
            pltpu.sync_copy(rows[b], acc_sh.at[dstw.at[t]], add=True)

    plsc.subcore_barrier()

    pltpu.sync_copy(acc_sh.at[pl.ds(r0, ROWS_PER_TILE)],
                    part_hbm.at[c, pl.ds(r0, ROWS_PER_TILE)])


_segsum_sc = pl.kernel(
    _segsum_body,
    out_type=jax.ShapeDtypeStruct((NC, NPAD, D), jnp.float32),
    mesh=plsc.VectorSubcoreMesh(core_axis_name="c", subcore_axis_name="s"),
    scratch_types=[
        pltpu.VMEM((WSZ, K), jnp.int32),
        pltpu.VMEM((WSZ, K), jnp.int32),
        pltpu.VMEM((K, D), jnp.float32),
        pltpu.VMEM((K, D), jnp.float32),
        pltpu.VMEM_SHARED((NPAD, D), jnp.float32),
        pltpu.SemaphoreType.DMA,
        pltpu.SemaphoreType.DMA,
        pltpu.SemaphoreType.DMA,
    ],
)


def _deg_body(srcs_hbm, zer_hbm, ones_hbm,
              degpart_hbm,
              src_v, ones_v, deg_sh):
    c = lax.axis_index("c")
    s = lax.axis_index("s")
    wid = s * NC + c

    r0 = s * ROWS_PER_TILE
    pltpu.sync_copy(zer_hbm.at[pl.ds(r0, ROWS_PER_TILE)],
                    deg_sh.at[pl.ds(r0, ROWS_PER_TILE)])
    pltpu.sync_copy(srcs_hbm.at[pl.ds(wid * CPT, CPT)], src_v)
    pltpu.sync_copy(ones_hbm, ones_v)

    plsc.subcore_barrier()

    @pl.loop(0, CPT)
    def _chunk(j):
        pltpu.sync_copy(ones_v, deg_sh.at[src_v.at[j]], add=True)

    plsc.subcore_barrier()

    pltpu.sync_copy(deg_sh.at[pl.ds(r0, ROWS_PER_TILE)],
                    degpart_hbm.at[c, pl.ds(r0, ROWS_PER_TILE)])


_deg_sc = pl.kernel(
    _deg_body,
    out_type=jax.ShapeDtypeStruct((NC, NPAD, DW), jnp.float32),
    mesh=plsc.VectorSubcoreMesh(core_axis_name="c", subcore_axis_name="s"),
    scratch_types=[
        pltpu.VMEM((CPT, K), jnp.int32),
        pltpu.VMEM((K, DW), jnp.float32),
        pltpu.VMEM_SHARED((NPAD, DW), jnp.float32),
    ],
)


ROWB = 1024
GRID = NPAD // ROWB


def _mat0_body(x_ref, w_ref, b_ref, o_ref):
    o_ref[...] = jnp.dot(x_ref[...], w_ref[...],
                         preferred_element_type=jnp.float32) + b_ref[...]


def _mat0(x, w, b):
    return pl.pallas_call(
        _mat0_body,
        grid=(GRID,),
        in_specs=[pl.BlockSpec((ROWB, D), lambda i: (i, 0)),
                  pl.BlockSpec((D, D), lambda i: (0, 0)),
                  pl.BlockSpec((1, D), lambda i: (0, 0))],
        out_specs=pl.BlockSpec((ROWB, D), lambda i: (i, 0)),
        out_shape=jax.ShapeDtypeStruct((NPAD, D), jnp.float32),
    )(x, w, b)


def _combine1_body(p0_ref, p1_ref, d0_ref, d1_ref, h_ref, g_ref):
    h = p0_ref[...] + p1_ref[...]
    deg = d0_ref[:, :1] + d1_ref[:, :1]
    inv = 1.0 / jnp.maximum(deg, 1.0)
    h_ref[...] = h
    g_ref[...] = h * inv


def _combine1(p0, p1, d0, d1):
    return pl.pallas_call(
        _combine1_body,
        grid=(GRID,),
        in_specs=[pl.BlockSpec((ROWB, D), lambda i: (i, 0)),
                  pl.BlockSpec((ROWB, D), lambda i: (i, 0)),
                  pl.BlockSpec((ROWB, DW), lambda i: (i, 0)),
                  pl.BlockSpec((ROWB, DW), lambda i: (i, 0))],
        out_specs=[pl.BlockSpec((ROWB, D), lambda i: (i, 0)),
                   pl.BlockSpec((ROWB, D), lambda i: (i, 0))],
        out_shape=[jax.ShapeDtypeStruct((NPAD, D), jnp.float32),
                   jax.ShapeDtypeStruct((NPAD, D), jnp.float32)],
    )(p0, p1, d0, d1)


def _layer_body(p0_ref, p1_ref, hp_ref, w_ref, b_ref, d0_ref, d1_ref,
                h_ref, g_ref):
    red = p0_ref[...] + p1_ref[...] + hp_ref[...]
    h = jnp.dot(red, w_ref[...], preferred_element_type=jnp.float32) + b_ref[...]
    deg = d0_ref[:, :1] + d1_ref[:, :1]
    inv = 1.0 / jnp.maximum(deg, 1.0)
    h_ref[...] = h
    g_ref[...] = h * inv


def _layer(p0, p1, hp, w, b, d0, d1):
    return pl.pallas_call(
        _layer_body,
        grid=(GRID,),
        in_specs=[pl.BlockSpec((ROWB, D), lambda i: (i, 0)),
                  pl.BlockSpec((ROWB, D), lambda i: (i, 0)),
                  pl.BlockSpec((ROWB, D), lambda i: (i, 0)),
                  pl.BlockSpec((D, D), lambda i: (0, 0)),
                  pl.BlockSpec((1, D), lambda i: (0, 0)),
                  pl.BlockSpec((ROWB, DW), lambda i: (i, 0)),
                  pl.BlockSpec((ROWB, DW), lambda i: (i, 0))],
        out_specs=[pl.BlockSpec((ROWB, D), lambda i: (i, 0)),
                   pl.BlockSpec((ROWB, D), lambda i: (i, 0))],
        out_shape=[jax.ShapeDtypeStruct((NPAD, D), jnp.float32),
                   jax.ShapeDtypeStruct((NPAD, D), jnp.float32)],
    )(p0, p1, hp, w, b, d0, d1)


def _final_body(h_ref, wp_ref, bp_ref, o_ref, acc_ref):
    i = pl.program_id(0)

    @pl.when(i == 0)
    def _():
        acc_ref[...] = jnp.zeros_like(acc_ref)

    row = lax.broadcasted_iota(jnp.int32, (ROWB, D), 0) + i * ROWB
    x = jnp.where(row < N, h_ref[...], 0.0)
    acc_ref[...] += jnp.sum(x, axis=0, keepdims=True)

    @pl.when(i == GRID - 1)
    def _():
        feat = acc_ref[...] * (1.0 / N)
        logits = jnp.dot(feat, wp_ref[...],
                         preferred_element_type=jnp.float32) + bp_ref[...]
        lane = lax.broadcasted_iota(jnp.int32, (1, D), 1)
        valid = lane < 16
        z = jnp.where(valid, logits, -1e30)
        z = z - jnp.max(z, axis=-1, keepdims=True)
        e = jnp.where(valid, jnp.exp(z), 0.0)
        o_ref[...] = e / jnp.sum(e, axis=-1, keepdims=True)


def _final(h, wp, bp):
    return pl.pallas_call(
        _final_body,
        grid=(GRID,),
        in_specs=[pl.BlockSpec((ROWB, D), lambda i: (i, 0)),
                  pl.BlockSpec((D, D), lambda i: (0, 0)),
                  pl.BlockSpec((1, D), lambda i: (0, 0))],
        out_specs=pl.BlockSpec((1, D), lambda i: (0, 0)),
        out_shape=jax.ShapeDtypeStruct((1, D), jnp.float32),
        scratch_shapes=[pltpu.VMEM((1, D), jnp.float32)],
    )(h, wp, bp)


def kernel(x, edge_feat, base_data, edge_index, Wn, bn, We, be, Wg, bg,
           Wgcn, bgcn, Wp, bp):
    src = edge_index[0].astype(jnp.int32)
    dst = edge_index[1].astype(jnp.int32)
    pad = jnp.full((EPAD - E,), N, jnp.int32)
    srcs = jnp.concatenate([src, pad]).reshape(NW * CPT, K)
    dsts = jnp.concatenate([dst, pad]).reshape(NW * CPT, K)

    zer = jnp.zeros((NPAD, D), jnp.float32)
    ones = jnp.ones((K, DW), jnp.float32)

    x_pad = jnp.pad(x, ((0, NPAD - N), (0, 0)))
    bn2 = bn.reshape(1, D)
    bg2 = bgcn.reshape(1, D)
    wp_pad = jnp.pad(Wp, ((0, 0), (0, D - 16)))
    bp_pad = jnp.pad(bp, ((0, D - 16))).reshape(1, D)

    h0 = _mat0(x_pad, Wn, bn2)

    degp = _deg_sc(srcs, zer, ones)
    part = _segsum_sc(h0, srcs, dsts, zer)
    h1, g = _combine1(part[0], part[1], degp[0], degp[1])

    h = h1
    for _ in range(3):
        part = _segsum_sc(g, srcs, dsts, zer)
        h, g = _layer(part[0], part[1], h, Wgcn, bg2, degp[0], degp[1])

    out = _final(h, wp_pad, bp_pad)
    return out[:, :16]


# speedup vs baseline: 1.1601x; 1.1601x over previous
import functools

import jax
import jax.numpy as jnp
from jax import lax
from jax.experimental import pallas as pl
from jax.experimental.pallas import tpu as pltpu
from jax.experimental.pallas import tpu_sc as plsc

N = 10000
E = 320000
D = 128
NPAD = 10240
K = 128
NC, NS = 2, 16
NW = NC * NS
CPT = 80
EPAD = NW * CPT * K
ROWS_PER_TILE = NPAD // NS
DW = 128


NB = 2
WSZ = 16
CPT0 = 128
CPT1 = 2 * CPT - CPT0


def _segsum_body(g_hbm, srcs_hbm, dsts_hbm, zer_hbm,
                 part_hbm,
                 srcw, dstw, rows0, rows1, acc_sh,
                 wsem, sem0, sem1):
    rows = (rows0, rows1)
    sems = (sem0, sem1)
    c = lax.axis_index("c")
    s = lax.axis_index("s")
    wid = s * NC + c

    r0 = s * ROWS_PER_TILE
    pltpu.sync_copy(zer_hbm.at[pl.ds(r0, ROWS_PER_TILE)],
                    acc_sh.at[pl.ds(r0, ROWS_PER_TILE)])

    plsc.subcore_barrier()

    def _start(t, b):
        pltpu.async_copy(g_hbm.at[srcw.at[t]], rows[b], sems[b])

    def _wait(b):
        pltpu.make_async_copy(g_hbm.at[pl.ds(0, K)], rows[b], sems[b]).wait()

    base = jnp.where(c == 0, s * CPT0, NS * CPT0 + s * CPT1)
    nwin = jnp.where(c == 0, CPT0 // WSZ, CPT1 // WSZ)

    @pl.loop(0, nwin)
    def _win(w):
        vec = base + w * WSZ + lax.iota(jnp.int32, WSZ)
        pltpu.async_copy(srcs_hbm.at[vec], srcw, wsem).wait()
        pltpu.async_copy(dsts_hbm.at[vec], dstw, wsem).wait()

        for b in range(NB):
            _start(b, b)

        @pl.loop(0, WSZ // NB - 1)
        def _group(g):
            for b in range(NB):
                t = g * NB + b
                _wait(b)
                pltpu.sync_copy(rows[b], acc_sh.at[dstw.at[t]], add=True)
                _start(t + NB, b)

        for b in range(NB):
            t = WSZ - NB + b
            _wait(b)
            pltpu.sync_copy(rows[b], acc_sh.at[dstw.at[t]], add=True)

    plsc.subcore_barrier()

    pltpu.sync_copy(acc_sh.at[pl.ds(r0, ROWS_PER_TILE)],
                    part_hbm.at[c, pl.ds(r0, ROWS_PER_TILE)])


_segsum_sc = pl.kernel(
    _segsum_body,
    out_type=jax.ShapeDtypeStruct((NC, NPAD, D), jnp.float32),
    mesh=plsc.VectorSubcoreMesh(core_axis_name="c", subcore_axis_name="s"),
    scratch_types=[
        pltpu.VMEM((WSZ, K), jnp.int32),
        pltpu.VMEM((WSZ, K), jnp.int32),
        pltpu.VMEM((K, D), jnp.float32),
        pltpu.VMEM((K, D), jnp.float32),
        pltpu.VMEM_SHARED((NPAD, D), jnp.float32),
        pltpu.SemaphoreType.DMA,
        pltpu.SemaphoreType.DMA,
        pltpu.SemaphoreType.DMA,
    ],
)


def _deg_body(srcs_hbm, zer_hbm, ones_hbm,
              degpart_hbm,
              src_v, ones_v, deg_sh):
    c = lax.axis_index("c")
    s = lax.axis_index("s")
    wid = s * NC + c

    r0 = s * ROWS_PER_TILE
    pltpu.sync_copy(zer_hbm.at[pl.ds(r0, ROWS_PER_TILE)],
                    deg_sh.at[pl.ds(r0, ROWS_PER_TILE)])
    pltpu.sync_copy(srcs_hbm.at[pl.ds(wid * CPT, CPT)], src_v)
    pltpu.sync_copy(ones_hbm, ones_v)

    plsc.subcore_barrier()

    @pl.loop(0, CPT)
    def _chunk(j):
        pltpu.sync_copy(ones_v, deg_sh.at[src_v.at[j]], add=True)

    plsc.subcore_barrier()

    pltpu.sync_copy(deg_sh.at[pl.ds(r0, ROWS_PER_TILE)],
                    degpart_hbm.at[c, pl.ds(r0, ROWS_PER_TILE)])


_deg_sc = pl.kernel(
    _deg_body,
    out_type=jax.ShapeDtypeStruct((NC, NPAD, DW), jnp.float32),
    mesh=plsc.VectorSubcoreMesh(core_axis_name="c", subcore_axis_name="s"),
    scratch_types=[
        pltpu.VMEM((CPT, K), jnp.int32),
        pltpu.VMEM((K, DW), jnp.float32),
        pltpu.VMEM_SHARED((NPAD, DW), jnp.float32),
    ],
)


ROWB = 1024
GRID = NPAD // ROWB


def _mat0_body(x_ref, w_ref, b_ref, o_ref):
    o_ref[...] = jnp.dot(x_ref[...], w_ref[...],
                         preferred_element_type=jnp.float32) + b_ref[...]


def _mat0(x, w, b):
    return pl.pallas_call(
        _mat0_body,
        grid=(GRID,),
        in_specs=[pl.BlockSpec((ROWB, D), lambda i: (i, 0)),
                  pl.BlockSpec((D, D), lambda i: (0, 0)),
                  pl.BlockSpec((1, D), lambda i: (0, 0))],
        out_specs=pl.BlockSpec((ROWB, D), lambda i: (i, 0)),
        out_shape=jax.ShapeDtypeStruct((NPAD, D), jnp.float32),
    )(x, w, b)


def _combine1_body(p0_ref, p1_ref, d0_ref, d1_ref, h_ref, g_ref):
    h = p0_ref[...] + p1_ref[...]
    deg = d0_ref[:, :1] + d1_ref[:, :1]
    inv = 1.0 / jnp.maximum(deg, 1.0)
    h_ref[...] = h
    g_ref[...] = h * inv


def _combine1(p0, p1, d0, d1):
    return pl.pallas_call(
        _combine1_body,
        grid=(GRID,),
        in_specs=[pl.BlockSpec((ROWB, D), lambda i: (i, 0)),
                  pl.BlockSpec((ROWB, D), lambda i: (i, 0)),
                  pl.BlockSpec((ROWB, DW), lambda i: (i, 0)),
                  pl.BlockSpec((ROWB, DW), lambda i: (i, 0))],
        out_specs=[pl.BlockSpec((ROWB, D), lambda i: (i, 0)),
                   pl.BlockSpec((ROWB, D), lambda i: (i, 0))],
        out_shape=[jax.ShapeDtypeStruct((NPAD, D), jnp.float32),
                   jax.ShapeDtypeStruct((NPAD, D), jnp.float32)],
    )(p0, p1, d0, d1)


def _layer_body(p0_ref, p1_ref, hp_ref, w_ref, b_ref, d0_ref, d1_ref,
                h_ref, g_ref):
    red = p0_ref[...] + p1_ref[...] + hp_ref[...]
    h = jnp.dot(red, w_ref[...], preferred_element_type=jnp.float32) + b_ref[...]
    deg = d0_ref[:, :1] + d1_ref[:, :1]
    inv = 1.0 / jnp.maximum(deg, 1.0)
    h_ref[...] = h
    g_ref[...] = h * inv


def _layer(p0, p1, hp, w, b, d0, d1):
    return pl.pallas_call(
        _layer_body,
        grid=(GRID,),
        in_specs=[pl.BlockSpec((ROWB, D), lambda i: (i, 0)),
                  pl.BlockSpec((ROWB, D), lambda i: (i, 0)),
                  pl.BlockSpec((ROWB, D), lambda i: (i, 0)),
                  pl.BlockSpec((D, D), lambda i: (0, 0)),
                  pl.BlockSpec((1, D), lambda i: (0, 0)),
                  pl.BlockSpec((ROWB, DW), lambda i: (i, 0)),
                  pl.BlockSpec((ROWB, DW), lambda i: (i, 0))],
        out_specs=[pl.BlockSpec((ROWB, D), lambda i: (i, 0)),
                   pl.BlockSpec((ROWB, D), lambda i: (i, 0))],
        out_shape=[jax.ShapeDtypeStruct((NPAD, D), jnp.float32),
                   jax.ShapeDtypeStruct((NPAD, D), jnp.float32)],
    )(p0, p1, hp, w, b, d0, d1)


def _final_body(h_ref, wp_ref, bp_ref, o_ref, acc_ref):
    i = pl.program_id(0)

    @pl.when(i == 0)
    def _():
        acc_ref[...] = jnp.zeros_like(acc_ref)

    row = lax.broadcasted_iota(jnp.int32, (ROWB, D), 0) + i * ROWB
    x = jnp.where(row < N, h_ref[...], 0.0)
    acc_ref[...] += jnp.sum(x, axis=0, keepdims=True)

    @pl.when(i == GRID - 1)
    def _():
        feat = acc_ref[...] * (1.0 / N)
        logits = jnp.dot(feat, wp_ref[...],
                         preferred_element_type=jnp.float32) + bp_ref[...]
        lane = lax.broadcasted_iota(jnp.int32, (1, D), 1)
        valid = lane < 16
        z = jnp.where(valid, logits, -1e30)
        z = z - jnp.max(z, axis=-1, keepdims=True)
        e = jnp.where(valid, jnp.exp(z), 0.0)
        o_ref[...] = e / jnp.sum(e, axis=-1, keepdims=True)


def _final(h, wp, bp):
    return pl.pallas_call(
        _final_body,
        grid=(GRID,),
        in_specs=[pl.BlockSpec((ROWB, D), lambda i: (i, 0)),
                  pl.BlockSpec((D, D), lambda i: (0, 0)),
                  pl.BlockSpec((1, D), lambda i: (0, 0))],
        out_specs=pl.BlockSpec((1, D), lambda i: (0, 0)),
        out_shape=jax.ShapeDtypeStruct((1, D), jnp.float32),
        scratch_shapes=[pltpu.VMEM((1, D), jnp.float32)],
    )(h, wp, bp)


def kernel(x, edge_feat, base_data, edge_index, Wn, bn, We, be, Wg, bg,
           Wgcn, bgcn, Wp, bp):
    src = edge_index[0].astype(jnp.int32)
    dst = edge_index[1].astype(jnp.int32)
    pad = jnp.full((EPAD - E,), N, jnp.int32)
    srcs = jnp.concatenate([src, pad]).reshape(NW * CPT, K)
    dsts = jnp.concatenate([dst, pad]).reshape(NW * CPT, K)

    zer = jnp.zeros((NPAD, D), jnp.float32)
    ones = jnp.ones((K, DW), jnp.float32)

    x_pad = jnp.pad(x, ((0, NPAD - N), (0, 0)))
    bn2 = bn.reshape(1, D)
    bg2 = bgcn.reshape(1, D)
    wp_pad = jnp.pad(Wp, ((0, 0), (0, D - 16)))
    bp_pad = jnp.pad(bp, ((0, D - 16))).reshape(1, D)

    h0 = _mat0(x_pad, Wn, bn2)

    degp = _deg_sc(srcs, zer, ones)
    part = _segsum_sc(h0, srcs, dsts, zer)
    h1, g = _combine1(part[0], part[1], degp[0], degp[1])

    h = h1
    for _ in range(3):
        part = _segsum_sc(g, srcs, dsts, zer)
        h, g = _layer(part[0], part[1], h, Wgcn, bg2, degp[0], degp[1])

    out = _final(h, wp_pad, bp_pad)
    return out[:, :16]
